# baseline (speedup 1.0000x reference)
import jax
import jax.numpy as jnp
from jax.experimental import pallas as pl

N_NODES = 10000
D_HID = 256
D_OUT = 16
E_EXP = 64
TOP_K = 2
D_FF = 512
CAP = 1024


def _gelu(x):
    return 0.5 * x * (1.0 + jax.lax.erf(x / jnp.sqrt(jnp.asarray(2.0, x.dtype))))


def _mm_kernel(x_ref, w_ref, o_ref):
    o_ref[...] = jnp.dot(x_ref[...], w_ref[...], preferred_element_type=jnp.float32)


def _mm(x, w, bm=1000):
    m, k = x.shape
    _, n = w.shape
    return pl.pallas_call(
        _mm_kernel,
        grid=(m // bm,),
        in_specs=[
            pl.BlockSpec((bm, k), lambda i: (i, 0)),
            pl.BlockSpec((k, n), lambda i: (0, 0)),
        ],
        out_specs=pl.BlockSpec((bm, n), lambda i: (i, 0)),
        out_shape=jax.ShapeDtypeStruct((m, n), jnp.float32),
    )(x, w)


def _ffn_kernel(x_ref, w1_ref, b1_ref, w2_ref, b2_ref, o_ref):
    xb = x_ref[0]
    mid = jnp.dot(xb, w1_ref[0], preferred_element_type=jnp.float32) + b1_ref[0]
    mid = _gelu(mid)
    o_ref[0] = jnp.dot(mid, w2_ref[0], preferred_element_type=jnp.float32) + b2_ref[0]


def _expert_ffn(buf, w1, b1, w2, b2, bc=512):
    e, cap, d = buf.shape
    b1r = b1.reshape(e, 1, D_FF)
    b2r = b2.reshape(e, 1, d)
    return pl.pallas_call(
        _ffn_kernel,
        grid=(e, cap // bc),
        in_specs=[
            pl.BlockSpec((1, bc, d), lambda i, j: (i, j, 0)),
            pl.BlockSpec((1, d, D_FF), lambda i, j: (i, 0, 0)),
            pl.BlockSpec((1, 1, D_FF), lambda i, j: (i, 0, 0)),
            pl.BlockSpec((1, D_FF, d), lambda i, j: (i, 0, 0)),
            pl.BlockSpec((1, 1, d), lambda i, j: (i, 0, 0)),
        ],
        out_specs=pl.BlockSpec((1, bc, d), lambda i, j: (i, j, 0)),
        out_shape=jax.ShapeDtypeStruct((e, cap, d), jnp.float32),
    )(buf, w1, b1r, w2, b2r)


def _moe_ff(h, p):
    nt, d = h.shape
    logits = _mm(h, p['gate_W']) + p['gate_b']
    topv, topi = jax.lax.top_k(logits, TOP_K)
    gate = jax.nn.softmax(topv, axis=-1)
    flat_e = topi.reshape(-1)
    counts = jnp.bincount(flat_e, length=E_EXP)
    starts = jnp.cumsum(counts) - counts
    order = jnp.argsort(flat_e)
    pos_sorted = jnp.arange(nt * TOP_K, dtype=jnp.int32) - starts[flat_e[order]].astype(jnp.int32)
    pos = jnp.zeros((nt * TOP_K,), jnp.int32).at[order].set(pos_sorted)
    valid = pos < CAP
    slot = jnp.where(valid, flat_e.astype(jnp.int32) * CAP + pos, E_EXP * CAP)
    x_rep = jnp.repeat(h, TOP_K, axis=0)
    buf = jnp.zeros((E_EXP * CAP + 1, d), h.dtype).at[slot].set(x_rep)
    eb = buf[:E_EXP * CAP].reshape(E_EXP, CAP, d)
    eout = _expert_ffn(eb, p['W1'], p['b1'], p['W2'], p['b2'])
    out_pad = jnp.concatenate([eout.reshape(E_EXP * CAP, d), jnp.zeros((1, d), h.dtype)], axis=0)
    gathered = out_pad[slot] * valid.astype(h.dtype)[:, None]
    core = jnp.sum(gathered.reshape(nt, TOP_K, d) * gate[:, :, None], axis=1)
    z = h + core
    mean = jnp.mean(z, axis=0)
    var = jnp.var(z, axis=0)
    zhat = (z - mean) / jnp.sqrt(var + 1e-5)
    return zhat * p['bn_g'] + p['bn_b']


def kernel(x, params, edge_index):
    src = edge_index[0]
    dst = edge_index[1]
    deg = jnp.bincount(dst, length=N_NODES).astype(jnp.float32) + 1.0
    dinv = 1.0 / jnp.sqrt(deg)
    coef = dinv[src] * dinv[dst]
    h = x
    for p in params['layers']:
        wcat = jnp.concatenate([p['Wg'], p['Wres']], axis=1)
        hw_res = _mm(h, wcat)
        hw = hw_res[:, :D_HID]
        res = hw_res[:, D_HID:]
        agg = jax.ops.segment_sum(hw[src] * coef[:, None], dst, num_segments=N_NODES)
        z = agg + hw / deg[:, None] + p['bg']
        z = z + res + p['bres']
        z = jax.nn.relu(z)
        h = _moe_ff(z, p)
    return _mm(h, params['final']['Wf']) + params['final']['bf']
